# baseline (device time: 25163 ns/iter reference)
import jax
import jax.numpy as jnp
from jax import lax
from jax.experimental import pallas as pl
from jax.experimental.pallas import tpu as pltpu

N_DEV = 8
_NEAR_FIRST = (1, 2, 4, 3, 5, 6, 7)
_FAR_FIRST = tuple(reversed(_NEAR_FIRST))


def _peer(my, o):
    b = my ^ ((my >> 1) & 1)
    q = b ^ o
    return q ^ ((q >> 1) & 1)


def kernel(x, dy):
    k, m = x.shape
    _, n = dy.shape
    m_out = m // N_DEV

    def body(x_ref, dy_ref, out_ref, pc_ref, recv_ref, send_sems, recv_sems):
        my = lax.axis_index("i")
        peers = [_peer(my, o) for o in _NEAR_FIRST]

        barrier = pltpu.get_barrier_semaphore()
        for d in peers:
            pl.semaphore_signal(
                barrier, inc=1, device_id=(d,),
                device_id_type=pl.DeviceIdType.MESH,
            )
        pl.semaphore_wait(barrier, N_DEV - 1)

        xb = x_ref[...].astype(jnp.bfloat16)
        db = dy_ref[...].astype(jnp.bfloat16)
        p = lax.dot_general(
            xb, db, (((0,), (0,)), ((), ())),
            preferred_element_type=jnp.float32,
        )
        pc_ref[...] = p.astype(jnp.bfloat16).reshape(N_DEV, m_out, n)

        sends = []
        for o in _FAR_FIRST:
            dst = _peer(my, o)
            rdma = pltpu.make_async_remote_copy(
                src_ref=pc_ref.at[dst],
                dst_ref=recv_ref.at[my],
                send_sem=send_sems.at[dst],
                recv_sem=recv_sems.at[my],
                device_id=(dst,),
                device_id_type=pl.DeviceIdType.MESH,
            )
            rdma.start()
            sends.append(rdma)

        out_ref[...] = pc_ref[pl.ds(my, 1)].reshape(m_out, n).astype(jnp.float32)

        for o in _NEAR_FIRST:
            s = _peer(my, o)
            recv = pltpu.make_async_remote_copy(
                src_ref=recv_ref.at[s],
                dst_ref=recv_ref.at[s],
                send_sem=send_sems.at[s],
                recv_sem=recv_sems.at[s],
                device_id=(s,),
                device_id_type=pl.DeviceIdType.MESH,
            )
            recv.wait_recv()
            out_ref[...] += recv_ref[pl.ds(s, 1)].reshape(m_out, n).astype(jnp.float32)

        for rdma in sends:
            rdma.wait_send()

    return pl.pallas_call(
        body,
        out_shape=jax.ShapeDtypeStruct((m_out, n), jnp.float32),
        in_specs=[
            pl.BlockSpec(memory_space=pltpu.VMEM),
            pl.BlockSpec(memory_space=pltpu.VMEM),
        ],
        out_specs=pl.BlockSpec(memory_space=pltpu.VMEM),
        scratch_shapes=[
            pltpu.VMEM((N_DEV, m_out, n), jnp.bfloat16),
            pltpu.VMEM((N_DEV, m_out, n), jnp.bfloat16),
            pltpu.SemaphoreType.DMA((N_DEV,)),
            pltpu.SemaphoreType.DMA((N_DEV,)),
        ],
        compiler_params=pltpu.CompilerParams(collective_id=0),
    )(x, dy)


# device time: 23592 ns/iter; 1.0666x vs baseline; 1.0666x over previous
import jax
import jax.numpy as jnp
from jax import lax
from jax.experimental import pallas as pl
from jax.experimental.pallas import tpu as pltpu

N_DEV = 8
N_HALF = 2


def kernel(x, dy):
    k, m = x.shape
    _, n = dy.shape
    m_out = m // N_DEV
    m_half = m // N_HALF
    c_half = N_DEV // N_HALF

    def body(x_ref, dy_ref, out_ref, pc_ref, recv_ref, send_sems, recv_sems):
        my = lax.axis_index("i")

        barrier = pltpu.get_barrier_semaphore()
        for d in range(N_DEV):
            @pl.when(my != d)
            def _():
                pl.semaphore_signal(
                    barrier, inc=1, device_id=(d,),
                    device_id_type=pl.DeviceIdType.MESH,
                )
        pl.semaphore_wait(barrier, N_DEV - 1)

        xb = x_ref[...].astype(jnp.bfloat16)
        db = dy_ref[...].astype(jnp.bfloat16)

        sends = []
        for h in range(N_HALF):
            ph = lax.dot_general(
                xb[:, h * m_half:(h + 1) * m_half], db,
                (((0,), (0,)), ((), ())),
                preferred_element_type=jnp.float32,
            )
            pc_ref[h * c_half:(h + 1) * c_half] = (
                ph.astype(jnp.bfloat16).reshape(c_half, m_out, n)
            )
            for c in range(h * c_half, (h + 1) * c_half):
                rdma = pltpu.make_async_remote_copy(
                    src_ref=pc_ref.at[c],
                    dst_ref=recv_ref.at[my],
                    send_sem=send_sems.at[c],
                    recv_sem=recv_sems.at[my],
                    device_id=(c,),
                    device_id_type=pl.DeviceIdType.MESH,
                )
                sends.append(rdma)

                @pl.when(my != c)
                def _():
                    rdma.start()

        recv_ref[pl.ds(my, 1)] = pc_ref[pl.ds(my, 1)]

        for s in range(N_DEV):
            recv = pltpu.make_async_remote_copy(
                src_ref=recv_ref.at[s],
                dst_ref=recv_ref.at[s],
                send_sem=send_sems.at[s],
                recv_sem=recv_sems.at[s],
                device_id=(s,),
                device_id_type=pl.DeviceIdType.MESH,
            )

            @pl.when(my != s)
            def _():
                recv.wait_recv()

        out_ref[...] = jnp.sum(recv_ref[...].astype(jnp.float32), axis=0)

        for c in range(N_DEV):
            @pl.when(my != c)
            def _():
                sends[c].wait_send()

    return pl.pallas_call(
        body,
        out_shape=jax.ShapeDtypeStruct((m_out, n), jnp.float32),
        in_specs=[
            pl.BlockSpec(memory_space=pltpu.VMEM),
            pl.BlockSpec(memory_space=pltpu.VMEM),
        ],
        out_specs=pl.BlockSpec(memory_space=pltpu.VMEM),
        scratch_shapes=[
            pltpu.VMEM((N_DEV, m_out, n), jnp.bfloat16),
            pltpu.VMEM((N_DEV, m_out, n), jnp.bfloat16),
            pltpu.SemaphoreType.DMA((N_DEV,)),
            pltpu.SemaphoreType.DMA((N_DEV,)),
        ],
        compiler_params=pltpu.CompilerParams(collective_id=0),
    )(x, dy)


# device time: 22891 ns/iter; 1.0993x vs baseline; 1.0306x over previous
import jax
import jax.numpy as jnp
from jax import lax
from jax.experimental import pallas as pl
from jax.experimental.pallas import tpu as pltpu

N_DEV = 8
N_HALF = 2


def kernel(x, dy):
    k, m = x.shape
    _, n = dy.shape
    m_out = m // N_DEV
    m_half = m // N_HALF
    c_half = N_DEV // N_HALF

    def body(x_ref, dy_ref, out_ref, pc_ref, recv_ref, send_sems, recv_sems):
        my = lax.axis_index("i")

        barrier = pltpu.get_barrier_semaphore()
        for d in range(N_DEV):
            @pl.when(my != d)
            def _():
                pl.semaphore_signal(
                    barrier, inc=1, device_id=(d,),
                    device_id_type=pl.DeviceIdType.MESH,
                )

        xb = x_ref[...].astype(jnp.bfloat16)
        db = dy_ref[...].astype(jnp.bfloat16)

        sends = []
        for h in range(N_HALF):
            ph = lax.dot_general(
                xb[:, h * m_half:(h + 1) * m_half], db,
                (((0,), (0,)), ((), ())),
                preferred_element_type=jnp.float32,
            )
            pc_ref[h * c_half:(h + 1) * c_half] = (
                ph.astype(jnp.bfloat16).reshape(c_half, m_out, n)
            )
            in_half = jnp.logical_and(h * c_half <= my, my < (h + 1) * c_half)

            @pl.when(in_half)
            def _():
                recv_ref[pl.ds(my, 1)] = pc_ref[pl.ds(my, 1)]

            if h == 0:
                pl.semaphore_wait(barrier, N_DEV - 1)

            for c in range(h * c_half, (h + 1) * c_half):
                rdma = pltpu.make_async_remote_copy(
                    src_ref=pc_ref.at[c],
                    dst_ref=recv_ref.at[my],
                    send_sem=send_sems.at[c],
                    recv_sem=recv_sems.at[my],
                    device_id=(c,),
                    device_id_type=pl.DeviceIdType.MESH,
                )
                sends.append(rdma)

                @pl.when(my != c)
                def _():
                    rdma.start()

        def wait_group(lo, hi):
            for s in range(lo, hi):
                recv = pltpu.make_async_remote_copy(
                    src_ref=recv_ref.at[s],
                    dst_ref=recv_ref.at[s],
                    send_sem=send_sems.at[s],
                    recv_sem=recv_sems.at[s],
                    device_id=(s,),
                    device_id_type=pl.DeviceIdType.MESH,
                )

                @pl.when(my != s)
                def _():
                    recv.wait_recv()

        wait_group(0, c_half)
        psum = jnp.sum(recv_ref[0:c_half].astype(jnp.float32), axis=0)
        wait_group(c_half, N_DEV)
        out_ref[...] = psum + jnp.sum(
            recv_ref[c_half:N_DEV].astype(jnp.float32), axis=0
        )

        for c in range(N_DEV):
            @pl.when(my != c)
            def _():
                sends[c].wait_send()

    return pl.pallas_call(
        body,
        out_shape=jax.ShapeDtypeStruct((m_out, n), jnp.float32),
        in_specs=[
            pl.BlockSpec(memory_space=pltpu.VMEM),
            pl.BlockSpec(memory_space=pltpu.VMEM),
        ],
        out_specs=pl.BlockSpec(memory_space=pltpu.VMEM),
        scratch_shapes=[
            pltpu.VMEM((N_DEV, m_out, n), jnp.bfloat16),
            pltpu.VMEM((N_DEV, m_out, n), jnp.bfloat16),
            pltpu.SemaphoreType.DMA((N_DEV,)),
            pltpu.SemaphoreType.DMA((N_DEV,)),
        ],
        compiler_params=pltpu.CompilerParams(collective_id=0),
    )(x, dy)


# device time: 21378 ns/iter; 1.1771x vs baseline; 1.0708x over previous
import jax
import jax.numpy as jnp
from jax import lax
from jax.experimental import pallas as pl
from jax.experimental.pallas import tpu as pltpu

N_DEV = 8
_RELAY = (
    (1, 6, 0, 768),
    (2, 5, 768, 768),
    (4, 3, 1536, 512),
)


def _peer(my, o):
    b = my ^ ((my >> 1) & 1)
    q = b ^ o
    return q ^ ((q >> 1) & 1)


def kernel(x, dy):
    k, m = x.shape
    _, n = dy.shape
    m_out = m // N_DEV

    def body(x_ref, dy_ref, out_ref, pc_ref, recv_ref, relay_ref,
             send_sems, recv_sems, rsend_sems, rrecv_sems):
        my = lax.axis_index("i")
        far = _peer(my, 7)

        barrier = pltpu.get_barrier_semaphore()
        for o in range(1, N_DEV):
            pl.semaphore_signal(
                barrier, inc=1, device_id=(_peer(my, o),),
                device_id_type=pl.DeviceIdType.MESH,
            )

        xb = x_ref[...].astype(jnp.bfloat16)
        db = dy_ref[...].astype(jnp.bfloat16)
        p = lax.dot_general(
            xb, db, (((0,), (0,)), ((), ())),
            preferred_element_type=jnp.float32,
        )
        pc_ref[...] = p.astype(jnp.bfloat16).reshape(N_DEV, m_out, n)

        pl.semaphore_wait(barrier, N_DEV - 1)

        relay_sends = []
        for di, (o, _, g0, gw) in enumerate(_RELAY):
            rdma = pltpu.make_async_remote_copy(
                src_ref=pc_ref.at[far, :, pl.ds(g0, gw)],
                dst_ref=relay_ref.at[di, :, pl.ds(0, gw)],
                send_sem=rsend_sems.at[di],
                recv_sem=rrecv_sems.at[di],
                device_id=(_peer(my, o),),
                device_id_type=pl.DeviceIdType.MESH,
            )
            rdma.start()
            relay_sends.append(rdma)

        sends = []
        for o in (1, 2, 4):
            dst = _peer(my, o)
            rdma = pltpu.make_async_remote_copy(
                src_ref=pc_ref.at[dst],
                dst_ref=recv_ref.at[my],
                send_sem=send_sems.at[dst],
                recv_sem=recv_sems.at[my],
                device_id=(dst,),
                device_id_type=pl.DeviceIdType.MESH,
            )
            rdma.start()
            sends.append(rdma)

        recv_ref[pl.ds(far, 1)] = jnp.zeros((1, m_out, n), jnp.bfloat16)
        recv_ref[pl.ds(my, 1)] = pc_ref[pl.ds(my, 1)]

        for di, (o, od, g0, gw) in enumerate(_RELAY):
            rrecv = pltpu.make_async_remote_copy(
                src_ref=relay_ref.at[di, :, pl.ds(0, gw)],
                dst_ref=relay_ref.at[di, :, pl.ds(0, gw)],
                send_sem=rsend_sems.at[di],
                recv_sem=rrecv_sems.at[di],
                device_id=(_peer(my, o),),
                device_id_type=pl.DeviceIdType.MESH,
            )
            rrecv.wait_recv()
            dst = _peer(my, od)
            pc_ref[pl.ds(dst, 1), :, pl.ds(g0, gw)] += (
                relay_ref[pl.ds(di, 1), :, pl.ds(0, gw)]
            )
            rdma = pltpu.make_async_remote_copy(
                src_ref=pc_ref.at[dst],
                dst_ref=recv_ref.at[my],
                send_sem=send_sems.at[dst],
                recv_sem=recv_sems.at[my],
                device_id=(dst,),
                device_id_type=pl.DeviceIdType.MESH,
            )
            rdma.start()
            sends.append(rdma)

        for o in (1, 2, 4, 3, 5, 6):
            s = _peer(my, o)
            recv = pltpu.make_async_remote_copy(
                src_ref=recv_ref.at[s],
                dst_ref=recv_ref.at[s],
                send_sem=send_sems.at[s],
                recv_sem=recv_sems.at[s],
                device_id=(s,),
                device_id_type=pl.DeviceIdType.MESH,
            )
            recv.wait_recv()

        out_ref[...] = jnp.sum(recv_ref[...].astype(jnp.float32), axis=0)

        for rdma in relay_sends + sends:
            rdma.wait_send()

    return pl.pallas_call(
        body,
        out_shape=jax.ShapeDtypeStruct((m_out, n), jnp.float32),
        in_specs=[
            pl.BlockSpec(memory_space=pltpu.VMEM),
            pl.BlockSpec(memory_space=pltpu.VMEM),
        ],
        out_specs=pl.BlockSpec(memory_space=pltpu.VMEM),
        scratch_shapes=[
            pltpu.VMEM((N_DEV, m_out, n), jnp.bfloat16),
            pltpu.VMEM((N_DEV, m_out, n), jnp.bfloat16),
            pltpu.VMEM((3, m_out, 768), jnp.bfloat16),
            pltpu.SemaphoreType.DMA((N_DEV,)),
            pltpu.SemaphoreType.DMA((N_DEV,)),
            pltpu.SemaphoreType.DMA((3,)),
            pltpu.SemaphoreType.DMA((3,)),
        ],
        compiler_params=pltpu.CompilerParams(collective_id=0),
    )(x, dy)
